# baseline (speedup 1.0000x reference)
import functools

import jax
import jax.numpy as jnp
from jax import lax
from jax.experimental import pallas as pl
from jax.experimental.pallas import tpu as pltpu
from jax.experimental.pallas import tpu_sc as plsc

N = 10000
E = 160000

NC = 2
NS = 16
L = 16
NW = NC * NS
NPT = 320
NPAD = NW * NPT
GB = 512
GC = 128
EPT_CAP = 16 * GB
EPAD = E + NW * GB + GB
HG = 16

RB = 512

_DIMS = [(128, 256), (256, 1024), (1024, 512), (512, 256), (256, 128)]


def _mm_body(x_ref, w_ref, o_ref):
    @pl.when(pl.program_id(2) == 0)
    def _():
        o_ref[...] = jnp.zeros_like(o_ref)
    o_ref[0] += jnp.dot(x_ref[0], w_ref[...], preferred_element_type=jnp.float32)


def _matmul(x, w, cin, cout):
    nrb = NPAD // RB
    return pl.pallas_call(
        _mm_body,
        grid=(nrb, cout, cin),
        in_specs=[
            pl.BlockSpec((1, RB, 128), lambda r, co, ci: (ci, r, 0)),
            pl.BlockSpec((128, 128), lambda r, co, ci: (ci, co)),
        ],
        out_specs=pl.BlockSpec((1, RB, 128), lambda r, co, ci: (co, r, 0)),
        out_shape=jax.ShapeDtypeStruct((cout, NPAD, 128), jnp.float32),
    )(x, w)


def _make_sd_body(c):
    def body(h_ref, as_ref, ad_ref, o_ref):
        dn = (((1,), (1,)), ((), ()))
        acc_s = jnp.zeros((1, RB), jnp.float32)
        acc_d = jnp.zeros((1, RB), jnp.float32)
        for i in range(c):
            hc = h_ref[i]
            acc_s += lax.dot_general(as_ref[i:i + 1], hc, dn,
                                     preferred_element_type=jnp.float32)
            acc_d += lax.dot_general(ad_ref[i:i + 1], hc, dn,
                                     preferred_element_type=jnp.float32)
        o_ref[...] = jnp.concatenate(
            [acc_s, acc_d, jnp.zeros((6, RB), jnp.float32)], axis=0)
    return body


def _sd(h, a_s, a_d, c):
    nrb = NPAD // RB
    return pl.pallas_call(
        _make_sd_body(c),
        grid=(nrb,),
        in_specs=[
            pl.BlockSpec((c, RB, 128), lambda r: (0, r, 0)),
            pl.BlockSpec((c, 128), lambda r: (0, 0)),
            pl.BlockSpec((c, 128), lambda r: (0, 0)),
        ],
        out_specs=pl.BlockSpec((8, RB), lambda r: (0, r)),
        out_shape=jax.ShapeDtypeStruct((8, NPAD), jnp.float32),
    )(h, a_s, a_d)


def _lsm_body(x_ref, o_ref):
    x = x_ref[...]
    mask = lax.broadcasted_iota(jnp.int32, x.shape, 1) < 40
    xm = jnp.where(mask, x, -jnp.inf)
    m = jnp.max(xm, axis=1, keepdims=True)
    ex = jnp.where(mask, jnp.exp(x - m), 0.0)
    s = jnp.sum(ex, axis=1, keepdims=True)
    o_ref[...] = (x - m) - jnp.log(s)


def _log_softmax(h):
    nrb = NPAD // RB
    return pl.pallas_call(
        _lsm_body,
        grid=(nrb,),
        in_specs=[pl.BlockSpec((RB, 128), lambda r: (r, 0))],
        out_specs=pl.BlockSpec((RB, 128), lambda r: (r, 0)),
        out_shape=jax.ShapeDtypeStruct((NPAD, 128), jnp.float32),
    )(h)


def _make_gat_sc(cout, relu):
    mesh = plsc.VectorSubcoreMesh(core_axis_name="c", subcore_axis_name="s")

    @functools.partial(
        pl.kernel,
        out_type=jax.ShapeDtypeStruct((cout * NPAD, 128), jnp.float32),
        mesh=mesh,
        compiler_params=pltpu.CompilerParams(needs_layout_passes=False),
        scratch_types=[
            pltpu.VMEM((NPAD,), jnp.float32),
            pltpu.VMEM((NPT + L,), jnp.float32),
            pltpu.VMEM((NPT + L,), jnp.float32),
            pltpu.VMEM((NPT + L,), jnp.float32),
            pltpu.VMEM((NPT + L,), jnp.float32),
            pltpu.VMEM((HG + GB + L,), jnp.int32),
            pltpu.VMEM((HG + GB + L,), jnp.int32),
            pltpu.VMEM((EPT_CAP,), jnp.float32),
            pltpu.VMEM((L + 8,), jnp.float32),
            pltpu.VMEM((EPT_CAP,), jnp.int32),
            pltpu.VMEM((EPT_CAP,), jnp.int32),
            pltpu.VMEM((GC,), jnp.int32),
            pltpu.VMEM((GC,), jnp.int32),
            pltpu.VMEM((GC,), jnp.int32),
            pltpu.VMEM((GC,), jnp.int32),
            pltpu.VMEM((GC, 128), jnp.float32),
            pltpu.VMEM((GC, 128), jnp.float32),
            pltpu.VMEM((64, 128), jnp.float32),
            pltpu.VMEM((128,), jnp.float32),
            pltpu.VMEM((NW * L,), jnp.int32),
            pltpu.VMEM((16, 128), jnp.float32),
            pltpu.VMEM_SHARED((NS * NPT, 128), jnp.float32),
            pltpu.SemaphoreType.DMA,
            pltpu.SemaphoreType.DMA,
        ],
    )
    def gat_sc(h_hbm, s_hbm, d_hbm, src_hbm, dst_hbm, meta_hbm, b_hbm, out_hbm,
               s_v, d_v, m_v, den_v, inv_v, src_st, dst_st, exb_v,
               hs_sc, srcall, dlall, src_c0, dl_c0, src_c1, dl_c1,
               rows0, rows1, bias_buf, b_v,
               meta_v, zbuf, shacc, sem0, sem1):
        cid = lax.axis_index("c")
        sid = lax.axis_index("s")
        wid = sid * NC + cid
        base_n = wid * NPT

        lane = lax.iota(jnp.int32, L)
        zf16 = jnp.zeros((L,), jnp.float32)
        neg = jnp.full((L,), -1e30, jnp.float32)
        sent = jnp.full((L,), -1, jnp.int32)

        pltpu.sync_copy(meta_hbm, meta_v)
        pltpu.sync_copy(s_hbm, s_v)
        pltpu.sync_copy(d_hbm.at[pl.ds(base_n, NPT)], d_v.at[pl.ds(0, NPT)])
        mrow = meta_v[pl.ds(pl.multiple_of(wid * L, L), L)]
        start = pl.multiple_of(mrow[0], GB)
        cnt = mrow[1]
        base_n = pl.multiple_of(base_n, NPT)

        for j in range((NPT + L) // L):
            m_v[pl.ds(j * L, L)] = neg
            den_v[pl.ds(j * L, L)] = zf16
        hs_sc[pl.ds(0, L)] = zf16

        nblk = (cnt + GB - 1) // GB

        def edge_chunk_vals(k, j):
            lo = HG + j * L
            valid = (k * GB + j * L + lane) < cnt
            srcs = src_st[pl.ds(lo, L)]
            dsts = dst_st[pl.ds(lo, L)]
            key_prev = dst_st[pl.ds(lo - 1, L)]
            key_next = dst_st[pl.ds(lo + 1, L)]
            srcs = jnp.where(valid, srcs, 0)
            dloc = jnp.where(valid, dsts - base_n, NPT)
            sv = plsc.load_gather(s_v, [srcs])
            dv = plsc.load_gather(d_v, [dloc])
            e = sv + dv
            e = jnp.where(e >= 0.0, e, 0.2 * e)
            e = jnp.where(valid, e, neg)
            isstart = key_prev != dsts
            lane_f = lane.astype(jnp.float32)
            sv_f = jnp.where(isstart, lane_f, 0.0)
            hs_sc[pl.ds(8, L)] = sv_f
            for sh in (1, 2, 4, 8):
                prev = plsc.load_gather(hs_sc, [lane + (8 - sh)])
                sv_f = jnp.maximum(sv_f, prev)
                if sh != 8:
                    hs_sc[pl.ds(8, L)] = sv_f
            sidx = sv_f.astype(jnp.int32)
            isend = ((key_next != dsts) | (lane == L - 1)) & valid
            return valid, dloc, e, sidx, isend

        def stage_blk(k):
            off = pl.multiple_of(start + k * GB, 8)
            pltpu.sync_copy(src_hbm.at[pl.ds(off, GB + L)],
                            src_st.at[pl.ds(HG, GB + L)])
            pltpu.sync_copy(dst_hbm.at[pl.ds(off, GB + L)],
                            dst_st.at[pl.ds(HG, GB + L)])

        def sweep1_blk(k, _):
            stage_blk(k)

            def chunk(j, _):
                valid, dloc, e, sidx, isend = edge_chunk_vals(k, j)
                hs_sc[pl.ds(8, L)] = e
                cur = e
                for sh in (1, 2, 4, 8):
                    prev = plsc.load_gather(hs_sc, [lane + (8 - sh)])
                    ok = (lane - sidx) >= sh
                    cur = jnp.where(ok, jnp.maximum(cur, prev), cur)
                    if sh != 8:
                        hs_sc[pl.ds(8, L)] = cur
                old = plsc.load_gather(m_v, [dloc], mask=isend)
                plsc.store_scatter(m_v, [dloc], jnp.maximum(old, cur),
                                   mask=isend)
                return 0
            lax.fori_loop(0, GB // L, chunk, 0)
            dst_st[pl.ds(0, L)] = dst_st[pl.ds(GB, L)]
            return 0

        def sweep2_blk(k, _):
            stage_blk(k)

            def chunk(j, _):
                valid, dloc, e, sidx, isend = edge_chunk_vals(k, j)
                mseg = plsc.load_gather(m_v, [dloc])
                ex = jnp.exp(e - mseg)
                ex = jnp.where(valid, ex, zf16)
                piece = ex
                hs_sc[pl.ds(8, L)] = piece
                for sh in (1, 2, 4, 8):
                    prev = plsc.load_gather(hs_sc, [lane + (8 - sh)])
                    ok = (lane - sidx) >= sh
                    piece = jnp.where(ok, piece + prev, piece)
                    if sh != 8:
                        hs_sc[pl.ds(8, L)] = piece
                old = plsc.load_gather(den_v, [dloc], mask=isend)
                plsc.store_scatter(den_v, [dloc], old + piece, mask=isend)
                exb_v[pl.ds(pl.multiple_of(k * GB + j * L, L), L)] = ex
                return 0
            lax.fori_loop(0, GB // L, chunk, 0)
            dst_st[pl.ds(0, L)] = dst_st[pl.ds(GB, L)]
            return 0

        dst_st[pl.ds(0, L)] = sent
        lax.fori_loop(0, nblk, sweep1_blk, 0)
        dst_st[pl.ds(0, L)] = sent
        lax.fori_loop(0, nblk, sweep2_blk, 0)

        for j in range((NPT + L) // L):
            den = den_v[pl.ds(j * L, L)]
            inv_v[pl.ds(j * L, L)] = 1.0 / (den + 1e-16)

        def zb_i(i, _):
            for f in range(8):
                zbuf[i, pl.ds(f * L, L)] = zf16
            return 0
        lax.fori_loop(0, 16, zb_i, 0)

        def load_all(kb, _):
            off = pl.multiple_of(start + kb * GB, 8)
            dsto = pl.multiple_of(kb * GB, 8)
            pltpu.sync_copy(src_hbm.at[pl.ds(off, GB)],
                            srcall.at[pl.ds(dsto, GB)])
            pltpu.sync_copy(dst_hbm.at[pl.ds(off, GB)],
                            dlall.at[pl.ds(dsto, GB)])
            return 0
        lax.fori_loop(0, nblk, load_all, 0)

        def prep_all(q, _):
            o = pl.multiple_of(q * L, L)
            valid = (q * L + lane) < cnt
            srcs = jnp.where(valid, srcall[pl.ds(o, L)], 0)
            dloc = jnp.where(valid, dlall[pl.ds(o, L)] - base_n, 0)
            ex = exb_v[pl.ds(o, L)]
            scl = ex * plsc.load_gather(inv_v, [jnp.where(valid, dloc, NPT)])
            srcall[pl.ds(o, L)] = srcs
            dlall[pl.ds(o, L)] = dloc + sid * NPT
            exb_v[pl.ds(o, L)] = jnp.where(valid, scl, zf16)
            return 0
        lax.fori_loop(0, nblk * (GB // L), prep_all, 0)

        npairs = nblk * 2

        def build_fire(k, srcb, dlb, rowsb, semb, c):
            ko = pl.multiple_of(k * GC, 8)
            for f in range(GC // L):
                srcb[pl.ds(f * L, L)] = srcall[pl.ds(ko + f * L, L)] + c * NPAD
                dlb[pl.ds(f * L, L)] = dlall[pl.ds(ko + f * L, L)]
            pltpu.async_copy(h_hbm.at[srcb], rowsb, semb)

        def scale_scatter(k, dlb, rowsb):
            kgc = k * GC

            def _scale(i, _):
                sp = plsc.load_gather(
                    exb_v, [jnp.zeros((L,), jnp.int32) + (kgc + i)])
                for f in range(8):
                    rowsb[i, pl.ds(f * L, L)] = rowsb[i, pl.ds(f * L, L)] * sp
                return 0
            lax.fori_loop(0, GC, _scale, 0)

            pltpu.sync_copy(rowsb, shacc.at[dlb], add=True)

        def per_c(c, _):
            pltpu.sync_copy(b_hbm.at[pl.ds(c * 128, 128)], b_v)

            def zero_i(i, _):
                pltpu.sync_copy(zbuf, shacc.at[pl.ds(sid * NPT + i * 16, 16)])
                return 0
            lax.fori_loop(0, NPT // 16, zero_i, 0)

            @pl.when(npairs > 0)
            def _():
                build_fire(0, src_c0, dl_c0, rows0, sem0, c)
                build_fire(1, src_c1, dl_c1, rows1, sem1, c)

            def pair(p, _):
                k0 = p * 2
                pltpu.make_async_copy(h_hbm.at[src_c0], rows0, sem0).wait()
                scale_scatter(k0, dl_c0, rows0)

                @pl.when(p + 1 < npairs)
                def _():
                    build_fire(k0 + 2, src_c0, dl_c0, rows0, sem0, c)

                pltpu.make_async_copy(h_hbm.at[src_c1], rows1, sem1).wait()
                scale_scatter(k0 + 1, dl_c1, rows1)

                @pl.when(p + 1 < npairs)
                def _():
                    build_fire(k0 + 3, src_c1, dl_c1, rows1, sem1, c)
                return 0
            lax.fori_loop(0, npairs, pair, 0)

            def bias_t(t, _):
                pltpu.sync_copy(
                    shacc.at[pl.ds(sid * NPT + t * 64, 64)], bias_buf)

                def bias_i(i, _):
                    for f in range(8):
                        v = bias_buf[i, pl.ds(f * L, L)] + b_v[pl.ds(f * L, L)]
                        if relu:
                            v = jnp.maximum(v, 0.0)
                        bias_buf[i, pl.ds(f * L, L)] = v
                    return 0
                lax.fori_loop(0, 64, bias_i, 0)

                pltpu.sync_copy(
                    bias_buf,
                    out_hbm.at[pl.ds(
                        pl.multiple_of(c * NPAD + base_n + t * 64, 64), 64)])
                return 0
            lax.fori_loop(0, NPT // 64, bias_t, 0)
            return 0
        lax.fori_loop(0, cout, per_c, 0)

    return gat_sc


def _prep_edges(edge_index):
    src = edge_index[0]
    dst = edge_index[1]
    order = jnp.argsort(dst)
    sdst = dst[order]
    ssrc = src[order]
    tile = sdst // NPT
    counts = jnp.bincount(tile, length=NW).astype(jnp.int32)
    acounts = ((counts + GB - 1) // GB) * GB
    astart = jnp.concatenate(
        [jnp.zeros((1,), jnp.int32), jnp.cumsum(acounts)[:-1].astype(jnp.int32)])
    cstart = jnp.concatenate(
        [jnp.zeros((1,), jnp.int32), jnp.cumsum(counts)[:-1].astype(jnp.int32)])
    pos = astart[tile] + (jnp.arange(E, dtype=jnp.int32) - cstart[tile])
    src_al = jnp.zeros((EPAD,), jnp.int32).at[pos].set(
        ssrc, indices_are_sorted=True, unique_indices=True)
    dst_al = jnp.full((EPAD,), -1, jnp.int32).at[pos].set(
        sdst, indices_are_sorted=True, unique_indices=True)
    meta = jnp.zeros((NW * L,), jnp.int32)
    meta = meta.at[jnp.arange(NW) * L].set(astart)
    meta = meta.at[jnp.arange(NW) * L + 1].set(counts)
    return src_al, dst_al, meta


def kernel(x, edge_index, W1, a_src1, a_dst1, b1, W2, a_src2, a_dst2, b2,
           W3, a_src3, a_dst3, b3, W4, a_src4, a_dst4, b4,
           W5, a_src5, a_dst5, b5):
    src_al, dst_al, meta = _prep_edges(edge_index)

    Ws = [W1, W2, W3, W4, jnp.pad(W5, ((0, 0), (0, 88)))]
    ass = [a_src1, a_src2, a_src3, a_src4, jnp.pad(a_src5, (0, 88))]
    ads = [a_dst1, a_dst2, a_dst3, a_dst4, jnp.pad(a_dst5, (0, 88))]
    bs = [b1, b2, b3, b4, jnp.pad(b5, (0, 88))]

    h = jnp.pad(x, ((0, NPAD - N), (0, 0))).reshape(1, NPAD, 128)
    for li, (di, do) in enumerate(_DIMS):
        cin, cout = di // 128, do // 128
        hw = _matmul(h, Ws[li], cin, cout)
        sd = _sd(hw, ass[li].reshape(cout, 128), ads[li].reshape(cout, 128),
                 cout)
        gat = _make_gat_sc(cout, relu=(li < 4))
        h = gat(hw.reshape(cout * NPAD, 128), sd[0], sd[1], src_al, dst_al,
                meta, bs[li]).reshape(cout, NPAD, 128)

    out = _log_softmax(h.reshape(NPAD, 128))
    return out[:N, :40]


# speedup vs baseline: 1.1166x; 1.1166x over previous
import functools

import jax
import jax.numpy as jnp
from jax import lax
from jax.experimental import pallas as pl
from jax.experimental.pallas import tpu as pltpu
from jax.experimental.pallas import tpu_sc as plsc

N = 10000
E = 160000

NC = 2
NS = 16
L = 16
NW = NC * NS
NPT = 320
NPAD = NW * NPT
GB = 512
GC = 256
EPT_CAP = 16 * GB
EPAD = E + NW * GB + GB
HG = 16

RB = 512

_DIMS = [(128, 256), (256, 1024), (1024, 512), (512, 256), (256, 128)]


def _mm_body(x_ref, w_ref, o_ref):
    @pl.when(pl.program_id(2) == 0)
    def _():
        o_ref[...] = jnp.zeros_like(o_ref)
    o_ref[0] += jnp.dot(x_ref[0], w_ref[...], preferred_element_type=jnp.float32)


def _matmul(x, w, cin, cout):
    nrb = NPAD // RB
    return pl.pallas_call(
        _mm_body,
        grid=(nrb, cout, cin),
        in_specs=[
            pl.BlockSpec((1, RB, 128), lambda r, co, ci: (ci, r, 0)),
            pl.BlockSpec((128, 128), lambda r, co, ci: (ci, co)),
        ],
        out_specs=pl.BlockSpec((1, RB, 128), lambda r, co, ci: (co, r, 0)),
        out_shape=jax.ShapeDtypeStruct((cout, NPAD, 128), jnp.float32),
    )(x, w)


def _make_sd_body(c):
    def body(h_ref, as_ref, ad_ref, o_ref):
        dn = (((1,), (1,)), ((), ()))
        acc_s = jnp.zeros((1, RB), jnp.float32)
        acc_d = jnp.zeros((1, RB), jnp.float32)
        for i in range(c):
            hc = h_ref[i]
            acc_s += lax.dot_general(as_ref[i:i + 1], hc, dn,
                                     preferred_element_type=jnp.float32)
            acc_d += lax.dot_general(ad_ref[i:i + 1], hc, dn,
                                     preferred_element_type=jnp.float32)
        o_ref[...] = jnp.concatenate(
            [acc_s, acc_d, jnp.zeros((6, RB), jnp.float32)], axis=0)
    return body


def _sd(h, a_s, a_d, c):
    nrb = NPAD // RB
    return pl.pallas_call(
        _make_sd_body(c),
        grid=(nrb,),
        in_specs=[
            pl.BlockSpec((c, RB, 128), lambda r: (0, r, 0)),
            pl.BlockSpec((c, 128), lambda r: (0, 0)),
            pl.BlockSpec((c, 128), lambda r: (0, 0)),
        ],
        out_specs=pl.BlockSpec((8, RB), lambda r: (0, r)),
        out_shape=jax.ShapeDtypeStruct((8, NPAD), jnp.float32),
    )(h, a_s, a_d)


def _lsm_body(x_ref, o_ref):
    x = x_ref[...]
    mask = lax.broadcasted_iota(jnp.int32, x.shape, 1) < 40
    xm = jnp.where(mask, x, -jnp.inf)
    m = jnp.max(xm, axis=1, keepdims=True)
    ex = jnp.where(mask, jnp.exp(x - m), 0.0)
    s = jnp.sum(ex, axis=1, keepdims=True)
    o_ref[...] = (x - m) - jnp.log(s)


def _log_softmax(h):
    nrb = NPAD // RB
    return pl.pallas_call(
        _lsm_body,
        grid=(nrb,),
        in_specs=[pl.BlockSpec((RB, 128), lambda r: (r, 0))],
        out_specs=pl.BlockSpec((RB, 128), lambda r: (r, 0)),
        out_shape=jax.ShapeDtypeStruct((NPAD, 128), jnp.float32),
    )(h)


def _make_gat_sc(cout, relu):
    mesh = plsc.VectorSubcoreMesh(core_axis_name="c", subcore_axis_name="s")

    @functools.partial(
        pl.kernel,
        out_type=jax.ShapeDtypeStruct((cout * NPAD, 128), jnp.float32),
        mesh=mesh,
        compiler_params=pltpu.CompilerParams(needs_layout_passes=False),
        scratch_types=[
            pltpu.VMEM((NPAD,), jnp.float32),
            pltpu.VMEM((NPT + L,), jnp.float32),
            pltpu.VMEM((NPT + L,), jnp.float32),
            pltpu.VMEM((NPT + L,), jnp.float32),
            pltpu.VMEM((NPT + L,), jnp.float32),
            pltpu.VMEM((HG + GB + L,), jnp.int32),
            pltpu.VMEM((HG + GB + L,), jnp.int32),
            pltpu.VMEM((EPT_CAP,), jnp.float32),
            pltpu.VMEM((L + 8,), jnp.float32),
            pltpu.VMEM((GC,), jnp.int32),
            pltpu.VMEM((GC,), jnp.int32),
            pltpu.VMEM((GC,), jnp.float32),
            pltpu.VMEM((GC, 128), jnp.float32),
            pltpu.VMEM((64, 128), jnp.float32),
            pltpu.VMEM((128,), jnp.float32),
            pltpu.VMEM((NW * L,), jnp.int32),
            pltpu.VMEM((16, 128), jnp.float32),
            pltpu.VMEM_SHARED((NS * NPT, 128), jnp.float32),
            pltpu.SemaphoreType.DMA,
        ],
    )
    def gat_sc(h_hbm, s_hbm, d_hbm, src_hbm, dst_hbm, meta_hbm, b_hbm, out_hbm,
               s_v, d_v, m_v, den_v, inv_v, src_st, dst_st, exb_v,
               hs_sc, src_c, dl_c, scl_v, rows_v, bias_buf, b_v,
               meta_v, zbuf, shacc, sem):
        cid = lax.axis_index("c")
        sid = lax.axis_index("s")
        wid = sid * NC + cid
        base_n = wid * NPT

        lane = lax.iota(jnp.int32, L)
        zf16 = jnp.zeros((L,), jnp.float32)
        neg = jnp.full((L,), -1e30, jnp.float32)
        sent = jnp.full((L,), -1, jnp.int32)

        pltpu.sync_copy(meta_hbm, meta_v)
        pltpu.sync_copy(s_hbm, s_v)
        pltpu.sync_copy(d_hbm.at[pl.ds(base_n, NPT)], d_v.at[pl.ds(0, NPT)])
        mrow = meta_v[pl.ds(pl.multiple_of(wid * L, L), L)]
        start = pl.multiple_of(mrow[0], GB)
        cnt = mrow[1]
        base_n = pl.multiple_of(base_n, NPT)

        for j in range((NPT + L) // L):
            m_v[pl.ds(j * L, L)] = neg
            den_v[pl.ds(j * L, L)] = zf16
        hs_sc[pl.ds(0, L)] = zf16

        nblk = (cnt + GB - 1) // GB

        def edge_chunk_vals(k, j):
            lo = HG + j * L
            valid = (k * GB + j * L + lane) < cnt
            srcs = src_st[pl.ds(lo, L)]
            dsts = dst_st[pl.ds(lo, L)]
            key_prev = dst_st[pl.ds(lo - 1, L)]
            key_next = dst_st[pl.ds(lo + 1, L)]
            srcs = jnp.where(valid, srcs, 0)
            dloc = jnp.where(valid, dsts - base_n, NPT)
            sv = plsc.load_gather(s_v, [srcs])
            dv = plsc.load_gather(d_v, [dloc])
            e = sv + dv
            e = jnp.where(e >= 0.0, e, 0.2 * e)
            e = jnp.where(valid, e, neg)
            isstart = key_prev != dsts
            lane_f = lane.astype(jnp.float32)
            sv_f = jnp.where(isstart, lane_f, 0.0)
            hs_sc[pl.ds(8, L)] = sv_f
            for sh in (1, 2, 4, 8):
                prev = plsc.load_gather(hs_sc, [lane + (8 - sh)])
                sv_f = jnp.maximum(sv_f, prev)
                if sh != 8:
                    hs_sc[pl.ds(8, L)] = sv_f
            sidx = sv_f.astype(jnp.int32)
            isend = ((key_next != dsts) | (lane == L - 1)) & valid
            return valid, dloc, e, sidx, isend

        def stage_blk(k):
            off = pl.multiple_of(start + k * GB, 8)
            pltpu.sync_copy(src_hbm.at[pl.ds(off, GB + L)],
                            src_st.at[pl.ds(HG, GB + L)])
            pltpu.sync_copy(dst_hbm.at[pl.ds(off, GB + L)],
                            dst_st.at[pl.ds(HG, GB + L)])

        def sweep1_blk(k, _):
            stage_blk(k)

            def chunk(j, _):
                valid, dloc, e, sidx, isend = edge_chunk_vals(k, j)
                hs_sc[pl.ds(8, L)] = e
                cur = e
                for sh in (1, 2, 4, 8):
                    prev = plsc.load_gather(hs_sc, [lane + (8 - sh)])
                    ok = (lane - sidx) >= sh
                    cur = jnp.where(ok, jnp.maximum(cur, prev), cur)
                    if sh != 8:
                        hs_sc[pl.ds(8, L)] = cur
                old = plsc.load_gather(m_v, [dloc], mask=isend)
                plsc.store_scatter(m_v, [dloc], jnp.maximum(old, cur),
                                   mask=isend)
                return 0
            lax.fori_loop(0, GB // L, chunk, 0)
            dst_st[pl.ds(0, L)] = dst_st[pl.ds(GB, L)]
            return 0

        def sweep2_blk(k, _):
            stage_blk(k)

            def chunk(j, _):
                valid, dloc, e, sidx, isend = edge_chunk_vals(k, j)
                mseg = plsc.load_gather(m_v, [dloc])
                ex = jnp.exp(e - mseg)
                ex = jnp.where(valid, ex, zf16)
                piece = ex
                hs_sc[pl.ds(8, L)] = piece
                for sh in (1, 2, 4, 8):
                    prev = plsc.load_gather(hs_sc, [lane + (8 - sh)])
                    ok = (lane - sidx) >= sh
                    piece = jnp.where(ok, piece + prev, piece)
                    if sh != 8:
                        hs_sc[pl.ds(8, L)] = piece
                old = plsc.load_gather(den_v, [dloc], mask=isend)
                plsc.store_scatter(den_v, [dloc], old + piece, mask=isend)
                exb_v[pl.ds(pl.multiple_of(k * GB + j * L, L), L)] = ex
                return 0
            lax.fori_loop(0, GB // L, chunk, 0)
            dst_st[pl.ds(0, L)] = dst_st[pl.ds(GB, L)]
            return 0

        dst_st[pl.ds(0, L)] = sent
        lax.fori_loop(0, nblk, sweep1_blk, 0)
        dst_st[pl.ds(0, L)] = sent
        lax.fori_loop(0, nblk, sweep2_blk, 0)

        for j in range((NPT + L) // L):
            den = den_v[pl.ds(j * L, L)]
            inv_v[pl.ds(j * L, L)] = 1.0 / (den + 1e-16)

        def zb_i(i, _):
            for f in range(8):
                zbuf[i, pl.ds(f * L, L)] = zf16
            return 0
        lax.fori_loop(0, 16, zb_i, 0)

        nblk2 = (cnt + GC - 1) // GC

        def per_c(c, _):
            pltpu.sync_copy(b_hbm.at[pl.ds(c * 128, 128)], b_v)

            def zero_i(i, _):
                pltpu.sync_copy(zbuf, shacc.at[pl.ds(sid * NPT + i * 16, 16)])
                return 0
            lax.fori_loop(0, NPT // 16, zero_i, 0)

            def blk(k, _):
                off = pl.multiple_of(start + k * GC, 8)
                pltpu.sync_copy(src_hbm.at[pl.ds(off, GC)], src_c)
                pltpu.sync_copy(dst_hbm.at[pl.ds(off, GC)], dl_c)

                def prep_chunk(j, _):
                    lo = j * L
                    valid = (k * GC + lo + lane) < cnt
                    srcs = jnp.where(valid, src_c[pl.ds(lo, L)], 0)
                    dloc = jnp.where(valid, dl_c[pl.ds(lo, L)] - base_n, 0)
                    ex = exb_v[pl.ds(k * GC + lo, L)]
                    scl = ex * plsc.load_gather(
                        inv_v, [jnp.where(valid, dloc, NPT)])
                    src_c[pl.ds(lo, L)] = srcs + c * NPAD
                    dl_c[pl.ds(lo, L)] = dloc + sid * NPT
                    scl_v[pl.ds(lo, L)] = jnp.where(valid, scl, zf16)
                    return 0
                lax.fori_loop(0, GC // L, prep_chunk, 0)

                pltpu.async_copy(h_hbm.at[src_c], rows_v, sem).wait()

                def scale_i(i, _):
                    sp = plsc.load_gather(
                        scl_v, [jnp.zeros((L,), jnp.int32) + i])
                    for f in range(8):
                        rows_v[i, pl.ds(f * L, L)] = \
                            rows_v[i, pl.ds(f * L, L)] * sp
                    return 0
                lax.fori_loop(0, GC, scale_i, 0)

                pltpu.sync_copy(rows_v, shacc.at[dl_c], add=True)
                return 0
            lax.fori_loop(0, nblk2, blk, 0)

            def bias_t(t, _):
                pltpu.sync_copy(
                    shacc.at[pl.ds(sid * NPT + t * 64, 64)], bias_buf)

                def bias_i(i, _):
                    for f in range(8):
                        v = bias_buf[i, pl.ds(f * L, L)] + b_v[pl.ds(f * L, L)]
                        if relu:
                            v = jnp.maximum(v, 0.0)
                        bias_buf[i, pl.ds(f * L, L)] = v
                    return 0
                lax.fori_loop(0, 64, bias_i, 0)

                pltpu.sync_copy(
                    bias_buf,
                    out_hbm.at[pl.ds(
                        pl.multiple_of(c * NPAD + base_n + t * 64, 64), 64)])
                return 0
            lax.fori_loop(0, NPT // 64, bias_t, 0)
            return 0
        lax.fori_loop(0, cout, per_c, 0)

    return gat_sc


def _prep_edges(edge_index):
    src = edge_index[0]
    dst = edge_index[1]
    order = jnp.argsort(dst)
    sdst = dst[order]
    ssrc = src[order]
    tile = sdst // NPT
    counts = jnp.bincount(tile, length=NW).astype(jnp.int32)
    acounts = ((counts + GB - 1) // GB) * GB
    astart = jnp.concatenate(
        [jnp.zeros((1,), jnp.int32), jnp.cumsum(acounts)[:-1].astype(jnp.int32)])
    cstart = jnp.concatenate(
        [jnp.zeros((1,), jnp.int32), jnp.cumsum(counts)[:-1].astype(jnp.int32)])
    pos = astart[tile] + (jnp.arange(E, dtype=jnp.int32) - cstart[tile])
    src_al = jnp.zeros((EPAD,), jnp.int32).at[pos].set(
        ssrc, indices_are_sorted=True, unique_indices=True)
    dst_al = jnp.full((EPAD,), -1, jnp.int32).at[pos].set(
        sdst, indices_are_sorted=True, unique_indices=True)
    meta = jnp.zeros((NW * L,), jnp.int32)
    meta = meta.at[jnp.arange(NW) * L].set(astart)
    meta = meta.at[jnp.arange(NW) * L + 1].set(counts)
    return src_al, dst_al, meta


def kernel(x, edge_index, W1, a_src1, a_dst1, b1, W2, a_src2, a_dst2, b2,
           W3, a_src3, a_dst3, b3, W4, a_src4, a_dst4, b4,
           W5, a_src5, a_dst5, b5):
    src_al, dst_al, meta = _prep_edges(edge_index)

    Ws = [W1, W2, W3, W4, jnp.pad(W5, ((0, 0), (0, 88)))]
    ass = [a_src1, a_src2, a_src3, a_src4, jnp.pad(a_src5, (0, 88))]
    ads = [a_dst1, a_dst2, a_dst3, a_dst4, jnp.pad(a_dst5, (0, 88))]
    bs = [b1, b2, b3, b4, jnp.pad(b5, (0, 88))]

    h = jnp.pad(x, ((0, NPAD - N), (0, 0))).reshape(1, NPAD, 128)
    for li, (di, do) in enumerate(_DIMS):
        cin, cout = di // 128, do // 128
        hw = _matmul(h, Ws[li], cin, cout)
        sd = _sd(hw, ass[li].reshape(cout, 128), ads[li].reshape(cout, 128),
                 cout)
        gat = _make_gat_sc(cout, relu=(li < 4))
        h = gat(hw.reshape(cout * NPAD, 128), sd[0], sd[1], src_al, dst_al,
                meta, bs[li]).reshape(cout, NPAD, 128)

    out = _log_softmax(h.reshape(NPAD, 128))
    return out[:N, :40]


# speedup vs baseline: 1.1942x; 1.0695x over previous
import functools

import jax
import jax.numpy as jnp
from jax import lax
from jax.experimental import pallas as pl
from jax.experimental.pallas import tpu as pltpu
from jax.experimental.pallas import tpu_sc as plsc

N = 10000
E = 160000

NC = 2
NS = 16
L = 16
NW = NC * NS
NPT = 320
NPAD = NW * NPT
GB = 512
GC = 128
EPT_CAP = 16 * GB
EPAD = E + NW * GB + GB
HG = 16

RB = 512

_DIMS = [(128, 256), (256, 1024), (1024, 512), (512, 256), (256, 128)]


def _mm_body(x_ref, w_ref, o_ref):
    @pl.when(pl.program_id(2) == 0)
    def _():
        o_ref[...] = jnp.zeros_like(o_ref)
    o_ref[0] += jnp.dot(x_ref[0], w_ref[...], preferred_element_type=jnp.float32)


def _matmul(x, w, cin, cout):
    nrb = NPAD // RB
    return pl.pallas_call(
        _mm_body,
        grid=(nrb, cout, cin),
        in_specs=[
            pl.BlockSpec((1, RB, 128), lambda r, co, ci: (ci, r, 0)),
            pl.BlockSpec((128, 128), lambda r, co, ci: (ci, co)),
        ],
        out_specs=pl.BlockSpec((1, RB, 128), lambda r, co, ci: (co, r, 0)),
        out_shape=jax.ShapeDtypeStruct((cout, NPAD, 128), jnp.float32),
    )(x, w)


def _make_sd_body(c):
    def body(h_ref, as_ref, ad_ref, o_ref):
        dn = (((1,), (1,)), ((), ()))
        acc_s = jnp.zeros((1, RB), jnp.float32)
        acc_d = jnp.zeros((1, RB), jnp.float32)
        for i in range(c):
            hc = h_ref[i]
            acc_s += lax.dot_general(as_ref[i:i + 1], hc, dn,
                                     preferred_element_type=jnp.float32)
            acc_d += lax.dot_general(ad_ref[i:i + 1], hc, dn,
                                     preferred_element_type=jnp.float32)
        o_ref[...] = jnp.concatenate(
            [acc_s, acc_d, jnp.zeros((6, RB), jnp.float32)], axis=0)
    return body


def _sd(h, a_s, a_d, c):
    nrb = NPAD // RB
    return pl.pallas_call(
        _make_sd_body(c),
        grid=(nrb,),
        in_specs=[
            pl.BlockSpec((c, RB, 128), lambda r: (0, r, 0)),
            pl.BlockSpec((c, 128), lambda r: (0, 0)),
            pl.BlockSpec((c, 128), lambda r: (0, 0)),
        ],
        out_specs=pl.BlockSpec((8, RB), lambda r: (0, r)),
        out_shape=jax.ShapeDtypeStruct((8, NPAD), jnp.float32),
    )(h, a_s, a_d)


def _lsm_body(x_ref, o_ref):
    x = x_ref[...]
    mask = lax.broadcasted_iota(jnp.int32, x.shape, 1) < 40
    xm = jnp.where(mask, x, -jnp.inf)
    m = jnp.max(xm, axis=1, keepdims=True)
    ex = jnp.where(mask, jnp.exp(x - m), 0.0)
    s = jnp.sum(ex, axis=1, keepdims=True)
    o_ref[...] = (x - m) - jnp.log(s)


def _log_softmax(h):
    nrb = NPAD // RB
    return pl.pallas_call(
        _lsm_body,
        grid=(nrb,),
        in_specs=[pl.BlockSpec((RB, 128), lambda r: (r, 0))],
        out_specs=pl.BlockSpec((RB, 128), lambda r: (r, 0)),
        out_shape=jax.ShapeDtypeStruct((NPAD, 128), jnp.float32),
    )(h)


def _make_gat_sc(cout, relu):
    mesh = plsc.VectorSubcoreMesh(core_axis_name="c", subcore_axis_name="s")

    @functools.partial(
        pl.kernel,
        out_type=jax.ShapeDtypeStruct((cout * NPAD, 128), jnp.float32),
        mesh=mesh,
        compiler_params=pltpu.CompilerParams(needs_layout_passes=False),
        scratch_types=[
            pltpu.VMEM((NPAD,), jnp.float32),
            pltpu.VMEM((NPT + L,), jnp.float32),
            pltpu.VMEM((NPT + L,), jnp.float32),
            pltpu.VMEM((NPT + L,), jnp.float32),
            pltpu.VMEM((NPT + L,), jnp.float32),
            pltpu.VMEM((HG + GB + L,), jnp.int32),
            pltpu.VMEM((HG + GB + L,), jnp.int32),
            pltpu.VMEM((EPT_CAP,), jnp.float32),
            pltpu.VMEM((L + 8,), jnp.float32),
            pltpu.VMEM((GC,), jnp.int32),
            pltpu.VMEM((GC,), jnp.int32),
            pltpu.VMEM((GC,), jnp.float32),
            pltpu.VMEM((GC, 128), jnp.float32),
            pltpu.VMEM((64, 128), jnp.float32),
            pltpu.VMEM((128,), jnp.float32),
            pltpu.VMEM((NW * L,), jnp.int32),
            pltpu.VMEM((16, 128), jnp.float32),
            pltpu.VMEM_SHARED((NS * NPT, 128), jnp.float32),
            pltpu.SemaphoreType.DMA,
        ],
    )
    def gat_sc(h_hbm, s_hbm, d_hbm, src_hbm, dst_hbm, meta_hbm, b_hbm, out_hbm,
               s_v, d_v, m_v, den_v, inv_v, src_st, dst_st, exb_v,
               hs_sc, src_c, dl_c, scl_v, rows_v, bias_buf, b_v,
               meta_v, zbuf, shacc, sem):
        cid = lax.axis_index("c")
        sid = lax.axis_index("s")
        wid = sid * NC + cid
        base_n = wid * NPT

        lane = lax.iota(jnp.int32, L)
        zf16 = jnp.zeros((L,), jnp.float32)
        neg = jnp.full((L,), -1e30, jnp.float32)
        sent = jnp.full((L,), -1, jnp.int32)

        pltpu.sync_copy(meta_hbm, meta_v)
        pltpu.sync_copy(s_hbm, s_v)
        pltpu.sync_copy(d_hbm.at[pl.ds(base_n, NPT)], d_v.at[pl.ds(0, NPT)])
        mrow = meta_v[pl.ds(pl.multiple_of(wid * L, L), L)]
        start = pl.multiple_of(mrow[0], GB)
        cnt = mrow[1]
        base_n = pl.multiple_of(base_n, NPT)

        for j in range((NPT + L) // L):
            m_v[pl.ds(j * L, L)] = neg
            den_v[pl.ds(j * L, L)] = zf16
        hs_sc[pl.ds(0, L)] = zf16

        nblk = (cnt + GB - 1) // GB

        def edge_chunk_vals(k, j):
            lo = HG + j * L
            valid = (k * GB + j * L + lane) < cnt
            srcs = src_st[pl.ds(lo, L)]
            dsts = dst_st[pl.ds(lo, L)]
            key_prev = dst_st[pl.ds(lo - 1, L)]
            key_next = dst_st[pl.ds(lo + 1, L)]
            srcs = jnp.where(valid, srcs, 0)
            dloc = jnp.where(valid, dsts - base_n, NPT)
            sv = plsc.load_gather(s_v, [srcs])
            dv = plsc.load_gather(d_v, [dloc])
            e = sv + dv
            e = jnp.where(e >= 0.0, e, 0.2 * e)
            e = jnp.where(valid, e, neg)
            isstart = key_prev != dsts
            lane_f = lane.astype(jnp.float32)
            sv_f = jnp.where(isstart, lane_f, 0.0)
            hs_sc[pl.ds(8, L)] = sv_f
            for sh in (1, 2, 4, 8):
                prev = plsc.load_gather(hs_sc, [lane + (8 - sh)])
                sv_f = jnp.maximum(sv_f, prev)
                if sh != 8:
                    hs_sc[pl.ds(8, L)] = sv_f
            sidx = sv_f.astype(jnp.int32)
            isend = ((key_next != dsts) | (lane == L - 1)) & valid
            return valid, dloc, e, sidx, isend

        def stage_blk(k):
            off = pl.multiple_of(start + k * GB, 8)
            pltpu.sync_copy(src_hbm.at[pl.ds(off, GB + L)],
                            src_st.at[pl.ds(HG, GB + L)])
            pltpu.sync_copy(dst_hbm.at[pl.ds(off, GB + L)],
                            dst_st.at[pl.ds(HG, GB + L)])

        def sweep1_blk(k, _):
            stage_blk(k)

            def chunk(j, _):
                valid, dloc, e, sidx, isend = edge_chunk_vals(k, j)
                hs_sc[pl.ds(8, L)] = e
                cur = e
                for sh in (1, 2, 4, 8):
                    prev = plsc.load_gather(hs_sc, [lane + (8 - sh)])
                    ok = (lane - sidx) >= sh
                    cur = jnp.where(ok, jnp.maximum(cur, prev), cur)
                    if sh != 8:
                        hs_sc[pl.ds(8, L)] = cur
                old = plsc.load_gather(m_v, [dloc], mask=isend)
                plsc.store_scatter(m_v, [dloc], jnp.maximum(old, cur),
                                   mask=isend)
                return 0
            lax.fori_loop(0, GB // L, chunk, 0)
            dst_st[pl.ds(0, L)] = dst_st[pl.ds(GB, L)]
            return 0

        def sweep2_blk(k, _):
            stage_blk(k)

            def chunk(j, _):
                valid, dloc, e, sidx, isend = edge_chunk_vals(k, j)
                mseg = plsc.load_gather(m_v, [dloc])
                ex = jnp.exp(e - mseg)
                ex = jnp.where(valid, ex, zf16)
                piece = ex
                hs_sc[pl.ds(8, L)] = piece
                for sh in (1, 2, 4, 8):
                    prev = plsc.load_gather(hs_sc, [lane + (8 - sh)])
                    ok = (lane - sidx) >= sh
                    piece = jnp.where(ok, piece + prev, piece)
                    if sh != 8:
                        hs_sc[pl.ds(8, L)] = piece
                old = plsc.load_gather(den_v, [dloc], mask=isend)
                plsc.store_scatter(den_v, [dloc], old + piece, mask=isend)
                exb_v[pl.ds(pl.multiple_of(k * GB + j * L, L), L)] = ex
                return 0
            lax.fori_loop(0, GB // L, chunk, 0)
            dst_st[pl.ds(0, L)] = dst_st[pl.ds(GB, L)]
            return 0

        dst_st[pl.ds(0, L)] = sent
        lax.fori_loop(0, nblk, sweep1_blk, 0)
        dst_st[pl.ds(0, L)] = sent
        lax.fori_loop(0, nblk, sweep2_blk, 0)

        for j in range((NPT + L) // L):
            den = den_v[pl.ds(j * L, L)]
            inv_v[pl.ds(j * L, L)] = 1.0 / (den + 1e-16)

        def zb_i(i, _):
            for f in range(8):
                zbuf[i, pl.ds(f * L, L)] = zf16
            return 0
        lax.fori_loop(0, 16, zb_i, 0)

        nblk2 = (cnt + GC - 1) // GC

        def per_c(c, _):
            pltpu.sync_copy(b_hbm.at[pl.ds(c * 128, 128)], b_v)

            def zero_i(i, _):
                pltpu.sync_copy(zbuf, shacc.at[pl.ds(sid * NPT + i * 16, 16)])
                return 0
            lax.fori_loop(0, NPT // 16, zero_i, 0)

            def blk(k, _):
                off = pl.multiple_of(start + k * GC, 8)
                pltpu.sync_copy(src_hbm.at[pl.ds(off, GC)], src_c)
                pltpu.sync_copy(dst_hbm.at[pl.ds(off, GC)], dl_c)

                def prep_chunk(j, _):
                    lo = j * L
                    valid = (k * GC + lo + lane) < cnt
                    srcs = jnp.where(valid, src_c[pl.ds(lo, L)], 0)
                    dloc = jnp.where(valid, dl_c[pl.ds(lo, L)] - base_n, 0)
                    ex = exb_v[pl.ds(k * GC + lo, L)]
                    scl = ex * plsc.load_gather(
                        inv_v, [jnp.where(valid, dloc, NPT)])
                    src_c[pl.ds(lo, L)] = srcs + c * NPAD
                    dl_c[pl.ds(lo, L)] = dloc + sid * NPT
                    scl_v[pl.ds(lo, L)] = jnp.where(valid, scl, zf16)
                    return 0
                lax.fori_loop(0, GC // L, prep_chunk, 0)

                pltpu.async_copy(h_hbm.at[src_c], rows_v, sem).wait()

                def scale_i(i, _):
                    sp = plsc.load_gather(
                        scl_v, [jnp.zeros((L,), jnp.int32) + i])
                    for f in range(8):
                        rows_v[i, pl.ds(f * L, L)] = \
                            rows_v[i, pl.ds(f * L, L)] * sp
                    return 0
                lax.fori_loop(0, GC, scale_i, 0)

                pltpu.sync_copy(rows_v, shacc.at[dl_c], add=True)
                return 0
            lax.fori_loop(0, nblk2, blk, 0)

            def bias_t(t, _):
                pltpu.sync_copy(
                    shacc.at[pl.ds(sid * NPT + t * 64, 64)], bias_buf)

                def bias_i(i, _):
                    for f in range(8):
                        v = bias_buf[i, pl.ds(f * L, L)] + b_v[pl.ds(f * L, L)]
                        if relu:
                            v = jnp.maximum(v, 0.0)
                        bias_buf[i, pl.ds(f * L, L)] = v
                    return 0
                lax.fori_loop(0, 64, bias_i, 0)

                pltpu.sync_copy(
                    bias_buf,
                    out_hbm.at[pl.ds(
                        pl.multiple_of(c * NPAD + base_n + t * 64, 64), 64)])
                return 0
            lax.fori_loop(0, NPT // 64, bias_t, 0)
            return 0
        lax.fori_loop(0, cout, per_c, 0)

    return gat_sc


def _prep_edges(edge_index):
    src = edge_index[0]
    dst = edge_index[1]
    sdst, ssrc = lax.sort_key_val(dst, src)
    tile = sdst // NPT
    counts = jnp.bincount(tile, length=NW).astype(jnp.int32)
    acounts = ((counts + GB - 1) // GB) * GB
    astart = jnp.concatenate(
        [jnp.zeros((1,), jnp.int32), jnp.cumsum(acounts)[:-1].astype(jnp.int32)])
    cstart = jnp.concatenate(
        [jnp.zeros((1,), jnp.int32), jnp.cumsum(counts)[:-1].astype(jnp.int32)])
    pos = astart[tile] + (jnp.arange(E, dtype=jnp.int32) - cstart[tile])
    src_al = jnp.zeros((EPAD,), jnp.int32).at[pos].set(
        ssrc, indices_are_sorted=True, unique_indices=True)
    dst_al = jnp.full((EPAD,), -1, jnp.int32).at[pos].set(
        sdst, indices_are_sorted=True, unique_indices=True)
    meta = jnp.zeros((NW * L,), jnp.int32)
    meta = meta.at[jnp.arange(NW) * L].set(astart)
    meta = meta.at[jnp.arange(NW) * L + 1].set(counts)
    return src_al, dst_al, meta


def kernel(x, edge_index, W1, a_src1, a_dst1, b1, W2, a_src2, a_dst2, b2,
           W3, a_src3, a_dst3, b3, W4, a_src4, a_dst4, b4,
           W5, a_src5, a_dst5, b5):
    src_al, dst_al, meta = _prep_edges(edge_index)

    Ws = [W1, W2, W3, W4, jnp.pad(W5, ((0, 0), (0, 88)))]
    ass = [a_src1, a_src2, a_src3, a_src4, jnp.pad(a_src5, (0, 88))]
    ads = [a_dst1, a_dst2, a_dst3, a_dst4, jnp.pad(a_dst5, (0, 88))]
    bs = [b1, b2, b3, b4, jnp.pad(b5, (0, 88))]

    h = jnp.pad(x, ((0, NPAD - N), (0, 0))).reshape(1, NPAD, 128)
    for li, (di, do) in enumerate(_DIMS):
        cin, cout = di // 128, do // 128
        hw = _matmul(h, Ws[li], cin, cout)
        sd = _sd(hw, ass[li].reshape(cout, 128), ads[li].reshape(cout, 128),
                 cout)
        gat = _make_gat_sc(cout, relu=(li < 4))
        h = gat(hw.reshape(cout * NPAD, 128), sd[0], sd[1], src_al, dst_al,
                meta, bs[li]).reshape(cout, NPAD, 128)

    out = _log_softmax(h.reshape(NPAD, 128))
    return out[:N, :40]


# speedup vs baseline: 1.2754x; 1.0680x over previous
import functools

import jax
import jax.numpy as jnp
from jax import lax
from jax.experimental import pallas as pl
from jax.experimental.pallas import tpu as pltpu
from jax.experimental.pallas import tpu_sc as plsc

N = 10000
E = 160000

NC = 2
NS = 16
L = 16
NW = NC * NS
NPT = 320
NPAD = NW * NPT
GB = 512
GC = 128
EPT_CAP = 16 * GB
EPAD = E + NW * GB + GB
HG = 16

RB = 512

_DIMS = [(128, 256), (256, 1024), (1024, 512), (512, 256), (256, 128)]


def _mm_body(x_ref, w_ref, o_ref):
    @pl.when(pl.program_id(2) == 0)
    def _():
        o_ref[...] = jnp.zeros_like(o_ref)
    o_ref[0] += jnp.dot(x_ref[0], w_ref[...], preferred_element_type=jnp.float32)


def _matmul(x, w, cin, cout):
    nrb = NPAD // RB
    return pl.pallas_call(
        _mm_body,
        grid=(nrb, cout, cin),
        in_specs=[
            pl.BlockSpec((1, RB, 128), lambda r, co, ci: (ci, r, 0)),
            pl.BlockSpec((128, 128), lambda r, co, ci: (ci, co)),
        ],
        out_specs=pl.BlockSpec((1, RB, 128), lambda r, co, ci: (co, r, 0)),
        out_shape=jax.ShapeDtypeStruct((cout, NPAD, 128), jnp.float32),
    )(x, w)


def _make_sd_body(c):
    def body(h_ref, as_ref, ad_ref, o_ref):
        dn = (((1,), (1,)), ((), ()))
        acc_s = jnp.zeros((1, RB), jnp.float32)
        acc_d = jnp.zeros((1, RB), jnp.float32)
        for i in range(c):
            hc = h_ref[i]
            acc_s += lax.dot_general(as_ref[i:i + 1], hc, dn,
                                     preferred_element_type=jnp.float32)
            acc_d += lax.dot_general(ad_ref[i:i + 1], hc, dn,
                                     preferred_element_type=jnp.float32)
        o_ref[...] = jnp.concatenate(
            [acc_s, acc_d, jnp.zeros((6, RB), jnp.float32)], axis=0)
    return body


def _sd(h, a_s, a_d, c):
    nrb = NPAD // RB
    return pl.pallas_call(
        _make_sd_body(c),
        grid=(nrb,),
        in_specs=[
            pl.BlockSpec((c, RB, 128), lambda r: (0, r, 0)),
            pl.BlockSpec((c, 128), lambda r: (0, 0)),
            pl.BlockSpec((c, 128), lambda r: (0, 0)),
        ],
        out_specs=pl.BlockSpec((8, RB), lambda r: (0, r)),
        out_shape=jax.ShapeDtypeStruct((8, NPAD), jnp.float32),
    )(h, a_s, a_d)


def _lsm_body(x_ref, o_ref):
    x = x_ref[...]
    mask = lax.broadcasted_iota(jnp.int32, x.shape, 1) < 40
    xm = jnp.where(mask, x, -jnp.inf)
    m = jnp.max(xm, axis=1, keepdims=True)
    ex = jnp.where(mask, jnp.exp(x - m), 0.0)
    s = jnp.sum(ex, axis=1, keepdims=True)
    o_ref[...] = (x - m) - jnp.log(s)


def _log_softmax(h):
    nrb = NPAD // RB
    return pl.pallas_call(
        _lsm_body,
        grid=(nrb,),
        in_specs=[pl.BlockSpec((RB, 128), lambda r: (r, 0))],
        out_specs=pl.BlockSpec((RB, 128), lambda r: (r, 0)),
        out_shape=jax.ShapeDtypeStruct((NPAD, 128), jnp.float32),
    )(h)


def _make_gat_sc(cout, relu):
    mesh = plsc.VectorSubcoreMesh(core_axis_name="c", subcore_axis_name="s")

    @functools.partial(
        pl.kernel,
        out_type=jax.ShapeDtypeStruct((cout * NPAD, 128), jnp.float32),
        mesh=mesh,
        compiler_params=pltpu.CompilerParams(needs_layout_passes=False),
        scratch_types=[
            pltpu.VMEM((NPAD,), jnp.float32),
            pltpu.VMEM((NPT + L,), jnp.float32),
            pltpu.VMEM((NPT + L,), jnp.float32),
            pltpu.VMEM((NPT + L,), jnp.float32),
            pltpu.VMEM((NPT + L,), jnp.float32),
            pltpu.VMEM((HG + GB + L,), jnp.int32),
            pltpu.VMEM((HG + GB + L,), jnp.int32),
            pltpu.VMEM((EPT_CAP,), jnp.float32),
            pltpu.VMEM((L + 8,), jnp.float32),
            pltpu.VMEM((GC,), jnp.int32),
            pltpu.VMEM((GC,), jnp.int32),
            pltpu.VMEM((GC,), jnp.float32),
            pltpu.VMEM((GC, 128), jnp.float32),
            pltpu.VMEM((64, 128), jnp.float32),
            pltpu.VMEM((128,), jnp.float32),
            pltpu.VMEM((NW * L,), jnp.int32),
            pltpu.VMEM((16, 128), jnp.float32),
            pltpu.VMEM_SHARED((NS * NPT, 128), jnp.float32),
            pltpu.SemaphoreType.DMA,
        ],
    )
    def gat_sc(h_hbm, s_hbm, d_hbm, src_hbm, dst_hbm, meta_hbm, b_hbm, out_hbm,
               s_v, d_v, m_v, den_v, inv_v, src_st, dst_st, exb_v,
               hs_sc, src_c, dl_c, scl_v, rows_v, bias_buf, b_v,
               meta_v, zbuf, shacc, sem):
        cid = lax.axis_index("c")
        sid = lax.axis_index("s")
        wid = sid * NC + cid
        base_n = wid * NPT

        lane = lax.iota(jnp.int32, L)
        zf16 = jnp.zeros((L,), jnp.float32)
        neg = jnp.full((L,), -1e30, jnp.float32)
        sent = jnp.full((L,), -1, jnp.int32)

        pltpu.sync_copy(meta_hbm, meta_v)
        pltpu.sync_copy(s_hbm, s_v)
        pltpu.sync_copy(d_hbm.at[pl.ds(base_n, NPT)], d_v.at[pl.ds(0, NPT)])
        mrow = meta_v[pl.ds(pl.multiple_of(wid * L, L), L)]
        start = pl.multiple_of(mrow[0], GB)
        cnt = mrow[1]
        base_n = pl.multiple_of(base_n, NPT)

        for j in range((NPT + L) // L):
            m_v[pl.ds(j * L, L)] = neg
            den_v[pl.ds(j * L, L)] = zf16
        hs_sc[pl.ds(0, L)] = zf16

        nblk = (cnt + GB - 1) // GB

        def edge_chunk_vals(k, j):
            lo = HG + j * L
            valid = (k * GB + j * L + lane) < cnt
            srcs = src_st[pl.ds(lo, L)]
            dsts = dst_st[pl.ds(lo, L)]
            key_prev = dst_st[pl.ds(lo - 1, L)]
            key_next = dst_st[pl.ds(lo + 1, L)]
            srcs = jnp.where(valid, srcs, 0)
            dloc = jnp.where(valid, dsts - base_n, NPT)
            sv = plsc.load_gather(s_v, [srcs])
            dv = plsc.load_gather(d_v, [dloc])
            e = sv + dv
            e = jnp.where(e >= 0.0, e, 0.2 * e)
            e = jnp.where(valid, e, neg)
            isstart = key_prev != dsts
            lane_f = lane.astype(jnp.float32)
            sv_f = jnp.where(isstart, lane_f, 0.0)
            hs_sc[pl.ds(8, L)] = sv_f
            for sh in (1, 2, 4, 8):
                prev = plsc.load_gather(hs_sc, [lane + (8 - sh)])
                sv_f = jnp.maximum(sv_f, prev)
                if sh != 8:
                    hs_sc[pl.ds(8, L)] = sv_f
            sidx = sv_f.astype(jnp.int32)
            isend = ((key_next != dsts) | (lane == L - 1)) & valid
            return valid, dloc, e, sidx, isend

        def stage_blk(k):
            off = pl.multiple_of(start + k * GB, 8)
            pltpu.sync_copy(src_hbm.at[pl.ds(off, GB + L)],
                            src_st.at[pl.ds(HG, GB + L)])
            pltpu.sync_copy(dst_hbm.at[pl.ds(off, GB + L)],
                            dst_st.at[pl.ds(HG, GB + L)])

        def sweep1_blk(k, _):
            stage_blk(k)

            def chunk(j, _):
                valid, dloc, e, sidx, isend = edge_chunk_vals(k, j)
                hs_sc[pl.ds(8, L)] = e
                cur = e
                for sh in (1, 2, 4, 8):
                    prev = plsc.load_gather(hs_sc, [lane + (8 - sh)])
                    ok = (lane - sidx) >= sh
                    cur = jnp.where(ok, jnp.maximum(cur, prev), cur)
                    if sh != 8:
                        hs_sc[pl.ds(8, L)] = cur
                old = plsc.load_gather(m_v, [dloc], mask=isend)
                plsc.store_scatter(m_v, [dloc], jnp.maximum(old, cur),
                                   mask=isend)
                return 0
            lax.fori_loop(0, GB // L, chunk, 0)
            dst_st[pl.ds(0, L)] = dst_st[pl.ds(GB, L)]
            return 0

        def sweep2_blk(k, _):
            stage_blk(k)

            def chunk(j, _):
                valid, dloc, e, sidx, isend = edge_chunk_vals(k, j)
                mseg = plsc.load_gather(m_v, [dloc])
                ex = jnp.exp(e - mseg)
                ex = jnp.where(valid, ex, zf16)
                piece = ex
                hs_sc[pl.ds(8, L)] = piece
                for sh in (1, 2, 4, 8):
                    prev = plsc.load_gather(hs_sc, [lane + (8 - sh)])
                    ok = (lane - sidx) >= sh
                    piece = jnp.where(ok, piece + prev, piece)
                    if sh != 8:
                        hs_sc[pl.ds(8, L)] = piece
                old = plsc.load_gather(den_v, [dloc], mask=isend)
                plsc.store_scatter(den_v, [dloc], old + piece, mask=isend)
                exb_v[pl.ds(pl.multiple_of(k * GB + j * L, L), L)] = ex
                return 0
            lax.fori_loop(0, GB // L, chunk, 0)
            dst_st[pl.ds(0, L)] = dst_st[pl.ds(GB, L)]
            return 0

        dst_st[pl.ds(0, L)] = sent
        lax.fori_loop(0, nblk, sweep1_blk, 0)
        dst_st[pl.ds(0, L)] = sent
        lax.fori_loop(0, nblk, sweep2_blk, 0)

        for j in range((NPT + L) // L):
            den = den_v[pl.ds(j * L, L)]
            inv_v[pl.ds(j * L, L)] = 1.0 / (den + 1e-16)

        def zb_i(i, _):
            for f in range(8):
                zbuf[i, pl.ds(f * L, L)] = zf16
            return 0
        lax.fori_loop(0, 16, zb_i, 0)

        nblk2 = (cnt + GC - 1) // GC

        def per_c(c, _):
            pltpu.sync_copy(b_hbm.at[pl.ds(c * 128, 128)], b_v)

            def zero_i(i, _):
                pltpu.sync_copy(zbuf, shacc.at[pl.ds(sid * NPT + i * 16, 16)])
                return 0
            lax.fori_loop(0, NPT // 16, zero_i, 0)

            def blk(k, _):
                off = pl.multiple_of(start + k * GC, 8)
                pltpu.sync_copy(src_hbm.at[pl.ds(off, GC)], src_c)
                pltpu.sync_copy(dst_hbm.at[pl.ds(off, GC)], dl_c)

                def prep_chunk(j, _):
                    lo = j * L
                    valid = (k * GC + lo + lane) < cnt
                    srcs = jnp.where(valid, src_c[pl.ds(lo, L)], 0)
                    dloc = jnp.where(valid, dl_c[pl.ds(lo, L)] - base_n, 0)
                    ex = exb_v[pl.ds(k * GC + lo, L)]
                    scl = ex * plsc.load_gather(
                        inv_v, [jnp.where(valid, dloc, NPT)])
                    src_c[pl.ds(lo, L)] = srcs + c * NPAD
                    dl_c[pl.ds(lo, L)] = dloc + sid * NPT
                    scl_v[pl.ds(lo, L)] = jnp.where(valid, scl, zf16)
                    return 0
                lax.fori_loop(0, GC // L, prep_chunk, 0)

                pltpu.async_copy(h_hbm.at[src_c], rows_v, sem).wait()

                @plsc.parallel_loop(0, GC, unroll=2)
                def _scale(i):
                    sp = plsc.load_gather(
                        scl_v, [jnp.zeros((L,), jnp.int32) + i])
                    for f in range(8):
                        rows_v[i, pl.ds(f * L, L)] = \
                            rows_v[i, pl.ds(f * L, L)] * sp

                pltpu.sync_copy(rows_v, shacc.at[dl_c], add=True)
                return 0
            lax.fori_loop(0, nblk2, blk, 0)

            def bias_t(t, _):
                pltpu.sync_copy(
                    shacc.at[pl.ds(sid * NPT + t * 64, 64)], bias_buf)

                def bias_i(i, _):
                    for f in range(8):
                        v = bias_buf[i, pl.ds(f * L, L)] + b_v[pl.ds(f * L, L)]
                        if relu:
                            v = jnp.maximum(v, 0.0)
                        bias_buf[i, pl.ds(f * L, L)] = v
                    return 0
                lax.fori_loop(0, 64, bias_i, 0)

                pltpu.sync_copy(
                    bias_buf,
                    out_hbm.at[pl.ds(
                        pl.multiple_of(c * NPAD + base_n + t * 64, 64), 64)])
                return 0
            lax.fori_loop(0, NPT // 64, bias_t, 0)
            return 0
        lax.fori_loop(0, cout, per_c, 0)

    return gat_sc


def _prep_edges(edge_index):
    src = edge_index[0]
    dst = edge_index[1]
    sdst, ssrc = lax.sort_key_val(dst, src)
    tile = sdst // NPT
    counts = jnp.bincount(tile, length=NW).astype(jnp.int32)
    acounts = ((counts + GB - 1) // GB) * GB
    astart = jnp.concatenate(
        [jnp.zeros((1,), jnp.int32), jnp.cumsum(acounts)[:-1].astype(jnp.int32)])
    cstart = jnp.concatenate(
        [jnp.zeros((1,), jnp.int32), jnp.cumsum(counts)[:-1].astype(jnp.int32)])
    pos = astart[tile] + (jnp.arange(E, dtype=jnp.int32) - cstart[tile])
    src_al = jnp.zeros((EPAD,), jnp.int32).at[pos].set(
        ssrc, indices_are_sorted=True, unique_indices=True)
    dst_al = jnp.full((EPAD,), -1, jnp.int32).at[pos].set(
        sdst, indices_are_sorted=True, unique_indices=True)
    meta = jnp.zeros((NW * L,), jnp.int32)
    meta = meta.at[jnp.arange(NW) * L].set(astart)
    meta = meta.at[jnp.arange(NW) * L + 1].set(counts)
    return src_al, dst_al, meta


def kernel(x, edge_index, W1, a_src1, a_dst1, b1, W2, a_src2, a_dst2, b2,
           W3, a_src3, a_dst3, b3, W4, a_src4, a_dst4, b4,
           W5, a_src5, a_dst5, b5):
    src_al, dst_al, meta = _prep_edges(edge_index)

    Ws = [W1, W2, W3, W4, jnp.pad(W5, ((0, 0), (0, 88)))]
    ass = [a_src1, a_src2, a_src3, a_src4, jnp.pad(a_src5, (0, 88))]
    ads = [a_dst1, a_dst2, a_dst3, a_dst4, jnp.pad(a_dst5, (0, 88))]
    bs = [b1, b2, b3, b4, jnp.pad(b5, (0, 88))]

    h = jnp.pad(x, ((0, NPAD - N), (0, 0))).reshape(1, NPAD, 128)
    for li, (di, do) in enumerate(_DIMS):
        cin, cout = di // 128, do // 128
        hw = _matmul(h, Ws[li], cin, cout)
        sd = _sd(hw, ass[li].reshape(cout, 128), ads[li].reshape(cout, 128),
                 cout)
        gat = _make_gat_sc(cout, relu=(li < 4))
        h = gat(hw.reshape(cout * NPAD, 128), sd[0], sd[1], src_al, dst_al,
                meta, bs[li]).reshape(cout, NPAD, 128)

    out = _log_softmax(h.reshape(NPAD, 128))
    return out[:N, :40]
